# Initial kernel scaffold; baseline (speedup 1.0000x reference)
import jax
import jax.numpy as jnp
from jax.experimental import pallas as pl


def kernel(user_emb, item_emb, W1, b1, W2, b2, trainUser, trainItem):
    raise NotImplementedError("write your pallas kernel here")


# speedup vs baseline: 6.8631x; 6.8631x over previous
import functools

import jax
import jax.numpy as jnp
from jax import lax
from jax.experimental import pallas as pl
from jax.experimental.pallas import tpu as pltpu
from jax.experimental.pallas import tpu_sc as plsc

NC = 2
NS = 16
L = 16
D = 128
NP = 5120
CHUNK = 80
RB = 64

f32 = jnp.float32
i32 = jnp.int32

_MESH = plsc.VectorSubcoreMesh(core_axis_name="c", subcore_axis_name="s")
ROWS = NP // NS


def _zero_fill(ref, nwords):
  z = jnp.zeros((L,), f32)
  def body(i, _):
    ref[pl.ds(i * L, L)] = z
    return ()
  lax.fori_loop(0, nwords // L, body, ())


def _zero_fill2d(ref, rows):
  z = jnp.zeros((L,), f32)
  def body(r, _):
    for m in range(D // L):
      ref[r, pl.ds(m * L, L)] = z
    return ()
  lax.fori_loop(0, rows, body, ())


def _offset_idx(idx_ref, off):
  def body(i, _):
    idx_ref[pl.ds(i * L, L)] = idx_ref[pl.ds(i * L, L)] + off
    return ()
  lax.fori_loop(0, CHUNK // L, body, ())


def _make_deg_kernel(E):
  Et = E // NS
  n_chunks = Et // CHUNK

  @functools.partial(
      pl.kernel,
      out_type=jax.ShapeDtypeStruct((2 * NP,), f32),
      mesh=_MESH,
      scratch_types=[
          pltpu.VMEM_SHARED((NP,), f32),
          pltpu.VMEM((CHUNK,), i32),
          pltpu.VMEM((CHUNK,), f32),
          pltpu.VMEM((ROWS,), f32),
      ],
  )
  def deg_kernel(dst_hbm, deg_hbm, hist_sh, idx_v, ones_v, slab_v):
    c = lax.axis_index("c")
    s = lax.axis_index("s")
    _zero_fill(slab_v, ROWS)
    pltpu.sync_copy(slab_v, hist_sh.at[pl.ds(s * ROWS, ROWS)])
    one = jnp.ones((L,), f32)
    for i in range(CHUNK // L):
      ones_v[pl.ds(i * L, L)] = one
    plsc.subcore_barrier()

    def body(ch, _):
      base = c * E + s * Et + ch * CHUNK
      pltpu.sync_copy(dst_hbm.at[pl.ds(base, CHUNK)], idx_v)
      pltpu.sync_copy(ones_v, hist_sh.at[idx_v], add=True)
      return ()
    lax.fori_loop(0, n_chunks, body, ())

    plsc.subcore_barrier()
    pltpu.sync_copy(hist_sh.at[pl.ds(s * ROWS, ROWS)], slab_v)
    pltpu.sync_copy(slab_v, deg_hbm.at[pl.ds((1 - c) * NP + s * ROWS, ROWS)])

  return deg_kernel


def _scale_body(deg, x, dinv, g):
  dv = lax.rsqrt(jnp.maximum(deg[:], 1.0))
  dinv[:] = dv
  g[:] = x[:] * dv[:, None]


def _scale_call(deg, x):
  return pl.pallas_call(
      _scale_body,
      out_shape=(jax.ShapeDtypeStruct((2 * NP,), f32),
                 jax.ShapeDtypeStruct((2 * NP, D), f32)),
  )(deg, x)


def _make_layer_kernel(E):
  Et = E // NS
  n_chunks = Et // CHUNK

  @functools.partial(
      pl.kernel,
      out_type=(jax.ShapeDtypeStruct((2 * NP, D), f32),
                jax.ShapeDtypeStruct((2 * NP, D), f32)),
      mesh=_MESH,
      scratch_types=[
          pltpu.VMEM_SHARED((NP, D), f32),
          pltpu.VMEM((CHUNK,), i32),
          pltpu.VMEM((CHUNK,), i32),
          pltpu.VMEM((CHUNK, D), f32),
          pltpu.VMEM((RB, D), f32),
          pltpu.VMEM((RB, D), f32),
          pltpu.VMEM((RB + L,), f32),
          pltpu.SemaphoreType.DMA,
      ],
  )
  def layer_kernel(g_hbm, dinv_hbm, acc_hbm, src_hbm, dst_hbm,
                   gn_hbm, accn_hbm,
                   out_sh, idxg_v, idxs_v, rows_v, vbuf, abuf, dv_v, sem):
    c = lax.axis_index("c")
    s = lax.axis_index("s")
    r0 = s * ROWS
    goff = c * NP
    noff = (1 - c) * NP

    _zero_fill2d(vbuf, RB)
    def zb(b, _):
      pltpu.sync_copy(vbuf, out_sh.at[pl.ds(r0 + b * RB, RB)])
      return ()
    lax.fori_loop(0, ROWS // RB, zb, ())
    plsc.subcore_barrier()

    def body(ch, _):
      base = c * E + s * Et + ch * CHUNK
      pltpu.sync_copy(src_hbm.at[pl.ds(base, CHUNK)], idxg_v)
      pltpu.sync_copy(dst_hbm.at[pl.ds(base, CHUNK)], idxs_v)
      _offset_idx(idxg_v, goff)
      pltpu.async_copy(g_hbm.at[idxg_v], rows_v, sem).wait()
      pltpu.sync_copy(rows_v, out_sh.at[idxs_v], add=True)
      return ()
    lax.fori_loop(0, n_chunks, body, ())

    plsc.subcore_barrier()

    def blk(b, _):
      rb0 = r0 + b * RB
      pltpu.sync_copy(out_sh.at[pl.ds(rb0, RB)], vbuf)
      pltpu.sync_copy(dinv_hbm.at[pl.ds(noff + rb0, RB)],
                      dv_v.at[pl.ds(0, RB)])
      pltpu.sync_copy(acc_hbm.at[pl.ds(noff + rb0, RB)], abuf)

      def rowf(r, _):
        d = dv_v[pl.ds(r, L)][0]
        for m in range(D // L):
          o = vbuf[r, pl.ds(m * L, L)]
          h = o * d
          abuf[r, pl.ds(m * L, L)] += h
          vbuf[r, pl.ds(m * L, L)] = h * d
        return ()
      lax.fori_loop(0, RB, rowf, ())
      pltpu.sync_copy(abuf, accn_hbm.at[pl.ds(noff + rb0, RB)])
      pltpu.sync_copy(vbuf, gn_hbm.at[pl.ds(noff + rb0, RB)])
      return ()
    lax.fori_loop(0, ROWS // RB, blk, ())

  return layer_kernel


def _pq_body(acc, w1a, w1b, b1, p, q):
  hp = jax.lax.Precision.HIGHEST
  ne = acc[:] * 0.25
  p[:] = jnp.dot(ne, w1a[:], precision=hp, preferred_element_type=f32) + b1[:]
  q[:] = jnp.dot(ne, w1b[:], precision=hp, preferred_element_type=f32)


def _pq_call(acc, w1a, w1b, b1):
  return pl.pallas_call(
      _pq_body,
      out_shape=(jax.ShapeDtypeStruct((2 * NP, D), f32),
                 jax.ShapeDtypeStruct((2 * NP, D), f32)),
  )(acc, w1a, w1b, b1)


def _make_logits_kernel(E):
  Et = E // NS
  n_chunks = Et // CHUNK

  @functools.partial(
      pl.kernel,
      out_type=jax.ShapeDtypeStruct((2 * E,), f32),
      mesh=_MESH,
      scratch_types=[
          pltpu.VMEM((CHUNK,), i32),
          pltpu.VMEM((CHUNK,), i32),
          pltpu.VMEM((CHUNK, D), f32),
          pltpu.VMEM((CHUNK, D), f32),
          pltpu.VMEM((D,), f32),
          pltpu.VMEM((L,), f32),
          pltpu.VMEM((CHUNK,), f32),
          pltpu.SemaphoreType.DMA,
          pltpu.SemaphoreType.DMA,
      ],
  )
  def logits_kernel(p_hbm, q_hbm, src_hbm, dst_hbm, w2_hbm, b2_hbm, out_hbm,
                    idxa_v, idxb_v, bufp, bufq, w2_v, b2_v, out_v,
                    sem1, sem2):
    c = lax.axis_index("c")
    s = lax.axis_index("s")
    goffa = c * NP
    goffb = (1 - c) * NP
    pltpu.sync_copy(w2_hbm, w2_v)
    pltpu.sync_copy(b2_hbm, b2_v)
    w2vecs = [w2_v[pl.ds(m * L, L)] for m in range(D // L)]
    b2s = b2_v[...][0]
    lane = lax.iota(i32, L)

    def body(ch, _):
      base = c * E + s * Et + ch * CHUNK
      pltpu.sync_copy(src_hbm.at[pl.ds(base, CHUNK)], idxa_v)
      pltpu.sync_copy(dst_hbm.at[pl.ds(base, CHUNK)], idxb_v)
      _offset_idx(idxa_v, goffa)
      _offset_idx(idxb_v, goffb)
      cp1 = pltpu.async_copy(p_hbm.at[idxa_v], bufp, sem1)
      cp2 = pltpu.async_copy(q_hbm.at[idxb_v], bufq, sem2)
      cp1.wait()
      cp2.wait()

      def group(jg, _):
        def edge(jj, ovec):
          j = jg * L + jj
          acc = jnp.zeros((L,), f32)
          for m in range(D // L):
            v = bufp[j, pl.ds(m * L, L)] + bufq[j, pl.ds(m * L, L)]
            v = jnp.maximum(v, 0.0)
            acc = acc + v * w2vecs[m]
          for sh in (8, 4, 2, 1):
            acc = acc + acc.at[lane ^ sh].get(mode="promise_in_bounds")
          return jnp.where(lane == jj, acc + b2s, ovec)
        ovec = lax.fori_loop(0, L, edge, jnp.zeros((L,), f32))
        out_v[pl.ds(jg * L, L)] = ovec
        return ()
      lax.fori_loop(0, CHUNK // L, group, ())
      pltpu.sync_copy(out_v, out_hbm.at[pl.ds(base, CHUNK)])
      return ()
    lax.fori_loop(0, n_chunks, body, ())

  return logits_kernel


def kernel(user_emb, item_emb, W1, b1, W2, b2, trainUser, trainItem):
  E = trainUser.shape[0]
  n_users = user_emb.shape[0]
  n_items = item_emb.shape[0]
  assert E % (NS * CHUNK) == 0 and n_users <= NP and n_items <= NP

  tu = trainUser.astype(i32)
  ti = trainItem.astype(i32)
  all_src = jnp.concatenate([tu, ti])
  all_dst = jnp.concatenate([ti, tu])
  xu = jnp.pad(user_emb, ((0, NP - n_users), (0, 0)))
  xi = jnp.pad(item_emb, ((0, NP - n_items), (0, 0)))
  x = jnp.concatenate([xu, xi], axis=0)

  deg = _make_deg_kernel(E)(all_dst)
  dinv, g = _scale_call(deg, x)

  layer = _make_layer_kernel(E)

  def _step(_, carry):
    cg, cacc = carry
    return tuple(layer(cg, dinv, cacc, all_src, all_dst))

  g, acc = lax.fori_loop(0, 3, _step, (g, x))

  p, q = _pq_call(acc, W1[:D], W1[D:], b1.reshape(1, D))

  w2 = W2[:, 0]
  b2p = jnp.pad(b2, (0, L - 1))
  logits = _make_logits_kernel(E)(p, q, all_src, all_dst, w2, b2p)

  x0 = jnp.concatenate([user_emb, item_emb], axis=0)
  src = jnp.concatenate([tu, ti + n_users])
  dst = jnp.concatenate([ti + n_users, tu])
  edge_index = jnp.stack([src, dst])
  return (lax.stop_gradient(x0), edge_index,
          lax.stop_gradient(logits.reshape(2 * E, 1)))


# speedup vs baseline: 11.3271x; 1.6504x over previous
import functools

import jax
import jax.numpy as jnp
from jax import lax
from jax.experimental import pallas as pl
from jax.experimental.pallas import tpu as pltpu
from jax.experimental.pallas import tpu_sc as plsc

NC = 2
NS = 16
L = 16
D = 128
NP = 5120
CHUNK = 80
RB = 16
NBUF = 5

f32 = jnp.float32
i32 = jnp.int32

_MESH = plsc.VectorSubcoreMesh(core_axis_name="c", subcore_axis_name="s")
ROWS = NP // NS


def _zero_fill(ref, nwords):
  z = jnp.zeros((L,), f32)
  def body(i, _):
    ref[pl.ds(i * L, L)] = z
    return ()
  lax.fori_loop(0, nwords // L, body, ())


def _zero_fill2d(ref, rows):
  z = jnp.zeros((L,), f32)
  def body(r, _):
    for m in range(D // L):
      ref[r, pl.ds(m * L, L)] = z
    return ()
  lax.fori_loop(0, rows, body, ())


def _offset_idx(idx_ref, off):
  def body(i, _):
    idx_ref[pl.ds(i * L, L)] = idx_ref[pl.ds(i * L, L)] + off
    return ()
  lax.fori_loop(0, CHUNK // L, body, ())


def _make_deg_kernel(E):
  Et = E // NS
  n_chunks = Et // CHUNK

  @functools.partial(
      pl.kernel,
      out_type=jax.ShapeDtypeStruct((2 * NP,), f32),
      mesh=_MESH,
      scratch_types=[
          pltpu.VMEM_SHARED((NP,), f32),
          pltpu.VMEM((CHUNK,), i32),
          pltpu.VMEM((CHUNK,), f32),
          pltpu.VMEM((ROWS,), f32),
      ],
  )
  def deg_kernel(dst_hbm, deg_hbm, hist_sh, idx_v, ones_v, slab_v):
    c = lax.axis_index("c")
    s = lax.axis_index("s")
    _zero_fill(slab_v, ROWS)
    pltpu.sync_copy(slab_v, hist_sh.at[pl.ds(s * ROWS, ROWS)])
    one = jnp.ones((L,), f32)
    for i in range(CHUNK // L):
      ones_v[pl.ds(i * L, L)] = one
    plsc.subcore_barrier()

    def body(ch, _):
      base = c * E + s * Et + ch * CHUNK
      pltpu.sync_copy(dst_hbm.at[pl.ds(base, CHUNK)], idx_v)
      pltpu.sync_copy(ones_v, hist_sh.at[idx_v], add=True)
      return ()
    lax.fori_loop(0, n_chunks, body, ())

    plsc.subcore_barrier()
    pltpu.sync_copy(hist_sh.at[pl.ds(s * ROWS, ROWS)], slab_v)
    pltpu.sync_copy(slab_v, deg_hbm.at[pl.ds((1 - c) * NP + s * ROWS, ROWS)])

  return deg_kernel


def _scale_body(deg, x, dinv, g):
  dv = lax.rsqrt(jnp.maximum(deg[:], 1.0))
  dinv[:] = dv
  g[:] = x[:] * dv[:, None]


def _scale_call(deg, x):
  return pl.pallas_call(
      _scale_body,
      out_shape=(jax.ShapeDtypeStruct((2 * NP,), f32),
                 jax.ShapeDtypeStruct((2 * NP, D), f32)),
  )(deg, x)


def _make_layer_kernel(E):
  Et = E // NS
  n_chunks = Et // CHUNK
  n_outer = n_chunks // NBUF
  assert n_chunks % NBUF == 0

  @functools.partial(
      pl.kernel,
      out_type=(jax.ShapeDtypeStruct((2 * NP, D), f32),
                jax.ShapeDtypeStruct((2 * NP, D), f32)),
      mesh=_MESH,
      scratch_types=[
          pltpu.VMEM_SHARED((NP, D), f32),
          pltpu.VMEM((n_chunks, CHUNK), i32),
          pltpu.VMEM((n_chunks, CHUNK), i32),
          pltpu.VMEM((NBUF, CHUNK, D), f32),
          pltpu.VMEM((RB, D), f32),
          pltpu.VMEM((RB, D), f32),
          pltpu.VMEM((RB + L,), f32),
      ] + [pltpu.SemaphoreType.DMA] * NBUF,
  )
  def layer_kernel(g_hbm, dinv_hbm, acc_hbm, src_hbm, dst_hbm,
                   gn_hbm, accn_hbm,
                   out_sh, idxg_v, idxs_v, ring, vbuf, abuf, dv_v, *sems):
    c = lax.axis_index("c")
    s = lax.axis_index("s")
    r0 = s * ROWS
    noff = (1 - c) * NP

    pltpu.sync_copy(src_hbm.at[c * NS + s], idxg_v)
    pltpu.sync_copy(dst_hbm.at[c * NS + s], idxs_v)

    _zero_fill2d(vbuf, RB)
    def zb(b, _):
      pltpu.sync_copy(vbuf, out_sh.at[pl.ds(r0 + b * RB, RB)])
      return ()
    lax.fori_loop(0, ROWS // RB, zb, ())
    plsc.subcore_barrier()

    for b in range(NBUF):
      pltpu.async_copy(g_hbm.at[idxg_v.at[b]], ring.at[b], sems[b])

    def outer(g, _):
      for b in range(NBUF):
        ch = g * NBUF + b
        pltpu.make_async_copy(g_hbm.at[pl.ds(0, CHUNK)], ring.at[b],
                              sems[b]).wait()
        pltpu.sync_copy(ring.at[b], out_sh.at[idxs_v.at[ch]], add=True)
        @pl.when(ch + NBUF < n_chunks)
        def _():
          pltpu.async_copy(g_hbm.at[idxg_v.at[ch + NBUF]], ring.at[b],
                           sems[b])
      return ()
    lax.fori_loop(0, n_outer, outer, ())

    plsc.subcore_barrier()

    def blk(b, _):
      rb0 = r0 + b * RB
      pltpu.sync_copy(out_sh.at[pl.ds(rb0, RB)], vbuf)
      pltpu.sync_copy(dinv_hbm.at[pl.ds(noff + rb0, RB)],
                      dv_v.at[pl.ds(0, RB)])
      pltpu.sync_copy(acc_hbm.at[pl.ds(noff + rb0, RB)], abuf)

      def rowf(r, _):
        d = dv_v[pl.ds(r, L)][0]
        for m in range(D // L):
          o = vbuf[r, pl.ds(m * L, L)]
          h = o * d
          abuf[r, pl.ds(m * L, L)] += h
          vbuf[r, pl.ds(m * L, L)] = h * d
        return ()
      lax.fori_loop(0, RB, rowf, ())
      pltpu.sync_copy(abuf, accn_hbm.at[pl.ds(noff + rb0, RB)])
      pltpu.sync_copy(vbuf, gn_hbm.at[pl.ds(noff + rb0, RB)])
      return ()
    lax.fori_loop(0, ROWS // RB, blk, ())

  return layer_kernel


def _pq_body(acc, w1a, w1b, b1, p, q):
  hp = jax.lax.Precision.HIGHEST
  ne = acc[:] * 0.25
  p[:] = jnp.dot(ne, w1a[:], precision=hp, preferred_element_type=f32) + b1[:]
  q[:] = jnp.dot(ne, w1b[:], precision=hp, preferred_element_type=f32)


def _pq_call(acc, w1a, w1b, b1):
  return pl.pallas_call(
      _pq_body,
      out_shape=(jax.ShapeDtypeStruct((2 * NP, D), f32),
                 jax.ShapeDtypeStruct((2 * NP, D), f32)),
  )(acc, w1a, w1b, b1)


def _make_logits_kernel(E):
  Et = E // NS
  n_chunks = Et // CHUNK

  @functools.partial(
      pl.kernel,
      out_type=jax.ShapeDtypeStruct((2 * E,), f32),
      mesh=_MESH,
      scratch_types=[
          pltpu.VMEM((CHUNK,), i32),
          pltpu.VMEM((CHUNK,), i32),
          pltpu.VMEM((CHUNK, D), f32),
          pltpu.VMEM((CHUNK, D), f32),
          pltpu.VMEM((D,), f32),
          pltpu.VMEM((L,), f32),
          pltpu.VMEM((CHUNK,), f32),
          pltpu.SemaphoreType.DMA,
          pltpu.SemaphoreType.DMA,
      ],
  )
  def logits_kernel(p_hbm, q_hbm, src_hbm, dst_hbm, w2_hbm, b2_hbm, out_hbm,
                    idxa_v, idxb_v, bufp, bufq, w2_v, b2_v, out_v,
                    sem1, sem2):
    c = lax.axis_index("c")
    s = lax.axis_index("s")
    pltpu.sync_copy(w2_hbm, w2_v)
    pltpu.sync_copy(b2_hbm, b2_v)
    w2vecs = [w2_v[pl.ds(m * L, L)] for m in range(D // L)]
    b2s = b2_v[...][0]
    lane = lax.iota(i32, L)

    def body(ch, _):
      base = c * E + s * Et + ch * CHUNK
      pltpu.sync_copy(src_hbm.at[pl.ds(base, CHUNK)], idxa_v)
      pltpu.sync_copy(dst_hbm.at[pl.ds(base, CHUNK)], idxb_v)
      cp1 = pltpu.async_copy(p_hbm.at[idxa_v], bufp, sem1)
      cp2 = pltpu.async_copy(q_hbm.at[idxb_v], bufq, sem2)
      cp1.wait()
      cp2.wait()

      def group(jg, _):
        def edge(jj, ovec):
          j = jg * L + jj
          acc = jnp.zeros((L,), f32)
          for m in range(D // L):
            v = bufp[j, pl.ds(m * L, L)] + bufq[j, pl.ds(m * L, L)]
            v = jnp.maximum(v, 0.0)
            acc = acc + v * w2vecs[m]
          for sh in (8, 4, 2, 1):
            acc = acc + acc.at[lane ^ sh].get(mode="promise_in_bounds")
          return jnp.where(lane == jj, acc + b2s, ovec)
        ovec = lax.fori_loop(0, L, edge, jnp.zeros((L,), f32))
        out_v[pl.ds(jg * L, L)] = ovec
        return ()
      lax.fori_loop(0, CHUNK // L, group, ())
      pltpu.sync_copy(out_v, out_hbm.at[pl.ds(base, CHUNK)])
      return ()
    lax.fori_loop(0, n_chunks, body, ())

  return logits_kernel


def kernel(user_emb, item_emb, W1, b1, W2, b2, trainUser, trainItem):
  E = trainUser.shape[0]
  n_users = user_emb.shape[0]
  n_items = item_emb.shape[0]
  assert E % (NS * CHUNK) == 0 and n_users <= NP and n_items <= NP

  tu = trainUser.astype(i32)
  ti = trainItem.astype(i32)
  all_dst = jnp.concatenate([ti, tu])
  n_ch = E // (NS * CHUNK)
  src_g2 = jnp.concatenate([tu, ti + NP]).reshape(NC * NS, n_ch, CHUNK)
  dst_l2 = all_dst.reshape(NC * NS, n_ch, CHUNK)
  xu = jnp.pad(user_emb, ((0, NP - n_users), (0, 0)))
  xi = jnp.pad(item_emb, ((0, NP - n_items), (0, 0)))
  x = jnp.concatenate([xu, xi], axis=0)

  deg = _make_deg_kernel(E)(all_dst)
  dinv, g = _scale_call(deg, x)

  layer = _make_layer_kernel(E)

  def _step(_, carry):
    cg, cacc = carry
    return tuple(layer(cg, dinv, cacc, src_g2, dst_l2))

  g, acc = lax.fori_loop(0, 3, _step, (g, x))

  p, q = _pq_call(acc, W1[:D], W1[D:], b1.reshape(1, D))

  w2 = W2[:, 0]
  b2p = jnp.pad(b2, (0, L - 1))
  srcg_1d = jnp.concatenate([tu, ti + NP])
  dstg_1d = jnp.concatenate([ti + NP, tu])
  logits = _make_logits_kernel(E)(p, q, srcg_1d, dstg_1d, w2, b2p)

  x0 = jnp.concatenate([user_emb, item_emb], axis=0)
  src = jnp.concatenate([tu, ti + n_users])
  dst = jnp.concatenate([ti + n_users, tu])
  edge_index = jnp.stack([src, dst])
  return (lax.stop_gradient(x0), edge_index,
          lax.stop_gradient(logits.reshape(2 * E, 1)))


# speedup vs baseline: 16.8397x; 1.4867x over previous
import functools

import jax
import jax.numpy as jnp
from jax import lax
from jax.experimental import pallas as pl
from jax.experimental.pallas import tpu as pltpu
from jax.experimental.pallas import tpu_sc as plsc

NC = 2
NS = 16
L = 16
D = 128
NP = 5120
CHUNK = 80
RB = 16
NBUF = 5

f32 = jnp.float32
i32 = jnp.int32

_MESH = plsc.VectorSubcoreMesh(core_axis_name="c", subcore_axis_name="s")
ROWS = NP // NS


def _zero_fill(ref, nwords):
  z = jnp.zeros((L,), f32)
  def body(i, _):
    ref[pl.ds(i * L, L)] = z
    return ()
  lax.fori_loop(0, nwords // L, body, ())


def _zero_fill2d(ref, rows):
  z = jnp.zeros((L,), f32)
  def body(r, _):
    for m in range(D // L):
      ref[r, pl.ds(m * L, L)] = z
    return ()
  lax.fori_loop(0, rows, body, ())


def _make_deg_kernel(E):
  Et = E // NS
  n_chunks = Et // CHUNK

  @functools.partial(
      pl.kernel,
      out_type=jax.ShapeDtypeStruct((2 * NP,), f32),
      mesh=_MESH,
      scratch_types=[
          pltpu.VMEM_SHARED((NP,), f32),
          pltpu.VMEM((n_chunks, CHUNK), i32),
          pltpu.VMEM((CHUNK,), f32),
          pltpu.VMEM((ROWS,), f32),
      ],
  )
  def deg_kernel(dst_hbm, deg_hbm, hist_sh, idx_v, ones_v, slab_v):
    c = lax.axis_index("c")
    s = lax.axis_index("s")
    pltpu.sync_copy(dst_hbm.at[c * NS + s], idx_v)
    _zero_fill(slab_v, ROWS)
    pltpu.sync_copy(slab_v, hist_sh.at[pl.ds(s * ROWS, ROWS)])
    one = jnp.ones((L,), f32)
    for i in range(CHUNK // L):
      ones_v[pl.ds(i * L, L)] = one
    plsc.subcore_barrier()

    def body(ch, _):
      pltpu.sync_copy(ones_v, hist_sh.at[idx_v.at[ch]], add=True)
      return ()
    lax.fori_loop(0, n_chunks, body, ())

    plsc.subcore_barrier()
    pltpu.sync_copy(hist_sh.at[pl.ds(s * ROWS, ROWS)], slab_v)
    pltpu.sync_copy(slab_v, deg_hbm.at[pl.ds((1 - c) * NP + s * ROWS, ROWS)])

  return deg_kernel


def _scale_body(deg, x, dinv, g):
  dv = lax.rsqrt(jnp.maximum(deg[:], 1.0))
  dinv[:] = dv
  g[:] = x[:] * dv[:, None]


def _scale_call(deg, x):
  return pl.pallas_call(
      _scale_body,
      out_shape=(jax.ShapeDtypeStruct((2 * NP,), f32),
                 jax.ShapeDtypeStruct((2 * NP, D), f32)),
  )(deg, x)


def _make_layer_kernel(E):
  Et = E // NS
  n_chunks = Et // CHUNK
  n_outer = n_chunks // NBUF
  assert n_chunks % NBUF == 0

  @functools.partial(
      pl.kernel,
      out_type=(jax.ShapeDtypeStruct((2 * NP, D), f32),
                jax.ShapeDtypeStruct((2 * NP, D), f32)),
      mesh=_MESH,
      scratch_types=[
          pltpu.VMEM_SHARED((NP, D), f32),
          pltpu.VMEM((n_chunks, CHUNK), i32),
          pltpu.VMEM((n_chunks, CHUNK), i32),
          pltpu.VMEM((NBUF, CHUNK, D), f32),
          pltpu.VMEM((RB, D), f32),
          pltpu.VMEM((RB, D), f32),
          pltpu.VMEM((RB + L,), f32),
      ] + [pltpu.SemaphoreType.DMA] * NBUF,
  )
  def layer_kernel(g_hbm, dinv_hbm, acc_hbm, src_hbm, dst_hbm,
                   gn_hbm, accn_hbm,
                   out_sh, idxg_v, idxs_v, ring, vbuf, abuf, dv_v, *sems):
    c = lax.axis_index("c")
    s = lax.axis_index("s")
    r0 = s * ROWS
    noff = (1 - c) * NP

    pltpu.sync_copy(src_hbm.at[c * NS + s], idxg_v)
    pltpu.sync_copy(dst_hbm.at[c * NS + s], idxs_v)

    _zero_fill2d(vbuf, RB)
    def zb(b, _):
      pltpu.sync_copy(vbuf, out_sh.at[pl.ds(r0 + b * RB, RB)])
      return ()
    lax.fori_loop(0, ROWS // RB, zb, ())
    plsc.subcore_barrier()

    for b in range(NBUF):
      pltpu.async_copy(g_hbm.at[idxg_v.at[b]], ring.at[b], sems[b])

    def outer(g, _):
      for b in range(NBUF):
        ch = g * NBUF + b
        pltpu.make_async_copy(g_hbm.at[pl.ds(0, CHUNK)], ring.at[b],
                              sems[b]).wait()
        pltpu.sync_copy(ring.at[b], out_sh.at[idxs_v.at[ch]], add=True)
        @pl.when(ch + NBUF < n_chunks)
        def _():
          pltpu.async_copy(g_hbm.at[idxg_v.at[ch + NBUF]], ring.at[b],
                           sems[b])
      return ()
    lax.fori_loop(0, n_outer, outer, ())

    plsc.subcore_barrier()

    def blk(b, _):
      rb0 = r0 + b * RB
      pltpu.sync_copy(out_sh.at[pl.ds(rb0, RB)], vbuf)
      pltpu.sync_copy(dinv_hbm.at[pl.ds(noff + rb0, RB)],
                      dv_v.at[pl.ds(0, RB)])
      pltpu.sync_copy(acc_hbm.at[pl.ds(noff + rb0, RB)], abuf)

      def rowf(r, _):
        d = dv_v[pl.ds(r, L)][0]
        for m in range(D // L):
          o = vbuf[r, pl.ds(m * L, L)]
          h = o * d
          abuf[r, pl.ds(m * L, L)] += h
          vbuf[r, pl.ds(m * L, L)] = h * d
        return ()
      lax.fori_loop(0, RB, rowf, ())
      pltpu.sync_copy(abuf, accn_hbm.at[pl.ds(noff + rb0, RB)])
      pltpu.sync_copy(vbuf, gn_hbm.at[pl.ds(noff + rb0, RB)])
      return ()
    lax.fori_loop(0, ROWS // RB, blk, ())

  return layer_kernel


def _pq_body(acc, w1a, w1b, b1, p, q):
  hp = jax.lax.Precision.HIGHEST
  ne = acc[:] * 0.25
  p[:] = jnp.dot(ne, w1a[:], precision=hp, preferred_element_type=f32) + b1[:]
  q[:] = jnp.dot(ne, w1b[:], precision=hp, preferred_element_type=f32)


def _pq_call(acc, w1a, w1b, b1):
  return pl.pallas_call(
      _pq_body,
      out_shape=(jax.ShapeDtypeStruct((2 * NP, D), f32),
                 jax.ShapeDtypeStruct((2 * NP, D), f32)),
  )(acc, w1a, w1b, b1)


def _make_logits_kernel(E):
  Et = E // NS
  n_chunks = Et // CHUNK
  n_outer = (n_chunks - 1) // 2

  @functools.partial(
      pl.kernel,
      out_type=jax.ShapeDtypeStruct((2 * E,), f32),
      mesh=_MESH,
      scratch_types=[
          pltpu.VMEM((n_chunks, CHUNK), i32),
          pltpu.VMEM((n_chunks, CHUNK), i32),
          pltpu.VMEM((2, CHUNK, D), f32),
          pltpu.VMEM((2, CHUNK, D), f32),
          pltpu.VMEM((D,), f32),
          pltpu.VMEM((L,), f32),
          pltpu.VMEM((Et,), f32),
          pltpu.SemaphoreType.DMA,
          pltpu.SemaphoreType.DMA,
          pltpu.SemaphoreType.DMA,
          pltpu.SemaphoreType.DMA,
      ],
  )
  def logits_kernel(p_hbm, q_hbm, src_hbm, dst_hbm, w2_hbm, b2_hbm, out_hbm,
                    idxa_v, idxb_v, bufp, bufq, w2_v, b2_v, out_v,
                    semp0, semq0, semp1, semq1):
    c = lax.axis_index("c")
    s = lax.axis_index("s")
    semp = (semp0, semp1)
    semq = (semq0, semq1)
    pltpu.sync_copy(src_hbm.at[c * NS + s], idxa_v)
    pltpu.sync_copy(dst_hbm.at[c * NS + s], idxb_v)
    pltpu.sync_copy(w2_hbm, w2_v)
    pltpu.sync_copy(b2_hbm, b2_v)
    w2vecs = [w2_v[pl.ds(m * L, L)] for m in range(D // L)]
    b2s = b2_v[...][0]
    lane = lax.iota(i32, L)

    def compute_chunk(ch, b):
      bp = bufp.at[b]
      bq = bufq.at[b]
      def group(jg, _):
        def edge(jj, ovec):
          j = jg * L + jj
          acc = jnp.zeros((L,), f32)
          for m in range(D // L):
            v = bp[j, pl.ds(m * L, L)] + bq[j, pl.ds(m * L, L)]
            v = jnp.maximum(v, 0.0)
            acc = acc + v * w2vecs[m]
          for sh in (8, 4, 2, 1):
            acc = acc + acc.at[lane ^ sh].get(mode="promise_in_bounds")
          return jnp.where(lane == jj, acc + b2s, ovec)
        ovec = lax.fori_loop(0, L, edge, jnp.zeros((L,), f32))
        out_v[pl.ds(ch * CHUNK + jg * L, L)] = ovec
        return ()
      lax.fori_loop(0, CHUNK // L, group, ())

    def issue(ch, b):
      pltpu.async_copy(p_hbm.at[idxa_v.at[ch]], bufp.at[b], semp[b])
      pltpu.async_copy(q_hbm.at[idxb_v.at[ch]], bufq.at[b], semq[b])

    def drain(b):
      pltpu.make_async_copy(p_hbm.at[pl.ds(0, CHUNK)], bufp.at[b],
                            semp[b]).wait()
      pltpu.make_async_copy(q_hbm.at[pl.ds(0, CHUNK)], bufq.at[b],
                            semq[b]).wait()

    issue(0, 0)
    issue(1, 1)

    def outer(g, _):
      for b in range(2):
        ch = g * 2 + b
        drain(b)
        compute_chunk(ch, b)
        nxt = ch + 2
        @pl.when(nxt < n_chunks)
        def _():
          issue(nxt, b)
      return ()
    lax.fori_loop(0, n_outer, outer, ())
    drain(0)
    compute_chunk(n_chunks - 1, 0)

    pltpu.sync_copy(out_v, out_hbm.at[pl.ds(c * E + s * Et, Et)])

  return logits_kernel


def kernel(user_emb, item_emb, W1, b1, W2, b2, trainUser, trainItem):
  E = trainUser.shape[0]
  n_users = user_emb.shape[0]
  n_items = item_emb.shape[0]
  assert E % (NS * CHUNK) == 0 and n_users <= NP and n_items <= NP

  tu = trainUser.astype(i32)
  ti = trainItem.astype(i32)
  all_dst = jnp.concatenate([ti, tu])
  n_ch = E // (NS * CHUNK)
  src_g2 = jnp.concatenate([tu, ti + NP]).reshape(NC * NS, n_ch, CHUNK)
  dst_l2 = all_dst.reshape(NC * NS, n_ch, CHUNK)
  xu = jnp.pad(user_emb, ((0, NP - n_users), (0, 0)))
  xi = jnp.pad(item_emb, ((0, NP - n_items), (0, 0)))
  x = jnp.concatenate([xu, xi], axis=0)

  deg = _make_deg_kernel(E)(dst_l2)
  dinv, g = _scale_call(deg, x)

  layer = _make_layer_kernel(E)

  def _step(_, carry):
    cg, cacc = carry
    return tuple(layer(cg, dinv, cacc, src_g2, dst_l2))

  g, acc = lax.fori_loop(0, 3, _step, (g, x))

  p, q = _pq_call(acc, W1[:D], W1[D:], b1.reshape(1, D))

  w2 = W2[:, 0]
  b2p = jnp.pad(b2, (0, L - 1))
  dst_g2 = jnp.concatenate([ti + NP, tu]).reshape(NC * NS, n_ch, CHUNK)
  logits = _make_logits_kernel(E)(p, q, src_g2, dst_g2, w2, b2p)

  x0 = jnp.concatenate([user_emb, item_emb], axis=0)
  src = jnp.concatenate([tu, ti + n_users])
  dst = jnp.concatenate([ti + n_users, tu])
  edge_index = jnp.stack([src, dst])
  return (lax.stop_gradient(x0), edge_index,
          lax.stop_gradient(logits.reshape(2 * E, 1)))


# speedup vs baseline: 19.5354x; 1.1601x over previous
import functools

import jax
import jax.numpy as jnp
from jax import lax
from jax.experimental import pallas as pl
from jax.experimental.pallas import tpu as pltpu
from jax.experimental.pallas import tpu_sc as plsc

NC = 2
NS = 16
L = 16
D = 128
NP = 5120
CHUNK = 80
RB = 32
NBUF = 5

f32 = jnp.float32
i32 = jnp.int32

_MESH = plsc.VectorSubcoreMesh(core_axis_name="c", subcore_axis_name="s")
ROWS = NP // NS


def _zero_fill(ref, nwords):
  z = jnp.zeros((L,), f32)
  def body(i, _):
    ref[pl.ds(i * L, L)] = z
    return ()
  lax.fori_loop(0, nwords // L, body, ())


def _zero_fill2d(ref, rows):
  z = jnp.zeros((L,), f32)
  def body(r, _):
    for m in range(D // L):
      ref[r, pl.ds(m * L, L)] = z
    return ()
  lax.fori_loop(0, rows, body, ())


def _make_deg_kernel(E):
  Et = E // NS
  n_chunks = Et // CHUNK

  @functools.partial(
      pl.kernel,
      out_type=jax.ShapeDtypeStruct((2 * NP,), f32),
      mesh=_MESH,
      scratch_types=[
          pltpu.VMEM_SHARED((NP,), f32),
          pltpu.VMEM((n_chunks, CHUNK), i32),
          pltpu.VMEM((CHUNK,), f32),
          pltpu.VMEM((ROWS,), f32),
      ],
  )
  def deg_kernel(dst_hbm, deg_hbm, hist_sh, idx_v, ones_v, slab_v):
    c = lax.axis_index("c")
    s = lax.axis_index("s")
    pltpu.sync_copy(dst_hbm.at[c * NS + s], idx_v)
    _zero_fill(slab_v, ROWS)
    pltpu.sync_copy(slab_v, hist_sh.at[pl.ds(s * ROWS, ROWS)])
    one = jnp.ones((L,), f32)
    for i in range(CHUNK // L):
      ones_v[pl.ds(i * L, L)] = one
    plsc.subcore_barrier()

    def body(ch, _):
      pltpu.sync_copy(ones_v, hist_sh.at[idx_v.at[ch]], add=True)
      return ()
    lax.fori_loop(0, n_chunks, body, ())

    plsc.subcore_barrier()
    pltpu.sync_copy(hist_sh.at[pl.ds(s * ROWS, ROWS)], slab_v)
    pltpu.sync_copy(slab_v, deg_hbm.at[pl.ds((1 - c) * NP + s * ROWS, ROWS)])

  return deg_kernel


def _scale_body(deg, x, dinv, g):
  dv = lax.rsqrt(jnp.maximum(deg[:], 1.0))
  dinv[:] = dv
  g[:] = x[:] * dv[:, None]


def _scale_call(deg, x):
  return pl.pallas_call(
      _scale_body,
      out_shape=(jax.ShapeDtypeStruct((2 * NP,), f32),
                 jax.ShapeDtypeStruct((2 * NP, D), f32)),
  )(deg, x)


def _make_layer_kernel(E):
  Et = E // NS
  n_chunks = Et // CHUNK
  n_outer = n_chunks // NBUF
  assert n_chunks % NBUF == 0

  @functools.partial(
      pl.kernel,
      out_type=jax.ShapeDtypeStruct((2 * NP, D), f32),
      mesh=_MESH,
      scratch_types=[
          pltpu.VMEM_SHARED((NP, D), f32),
          pltpu.VMEM((n_chunks, CHUNK), i32),
          pltpu.VMEM((n_chunks, CHUNK), i32),
          pltpu.VMEM((NBUF, CHUNK, D), f32),
          pltpu.VMEM((RB, D), f32),
          pltpu.VMEM((RB + L,), f32),
      ] + [pltpu.SemaphoreType.DMA] * NBUF,
  )
  def layer_kernel(g_hbm, dinv_hbm, src_hbm, dst_hbm, gn_hbm,
                   out_sh, idxg_v, idxs_v, ring, vbuf, dv_v, *sems):
    c = lax.axis_index("c")
    s = lax.axis_index("s")
    r0 = s * ROWS
    noff = (1 - c) * NP

    pltpu.sync_copy(src_hbm.at[c * NS + s], idxg_v)
    pltpu.sync_copy(dst_hbm.at[c * NS + s], idxs_v)

    _zero_fill2d(vbuf, RB)
    def zb(b, _):
      pltpu.sync_copy(vbuf, out_sh.at[pl.ds(r0 + b * RB, RB)])
      return ()
    lax.fori_loop(0, ROWS // RB, zb, ())
    plsc.subcore_barrier()

    for b in range(NBUF):
      pltpu.async_copy(g_hbm.at[idxg_v.at[b]], ring.at[b], sems[b])

    def outer(g, _):
      for b in range(NBUF):
        ch = g * NBUF + b
        pltpu.make_async_copy(g_hbm.at[pl.ds(0, CHUNK)], ring.at[b],
                              sems[b]).wait()
        pltpu.sync_copy(ring.at[b], out_sh.at[idxs_v.at[ch]], add=True)
        @pl.when(ch + NBUF < n_chunks)
        def _():
          pltpu.async_copy(g_hbm.at[idxg_v.at[ch + NBUF]], ring.at[b],
                           sems[b])
      return ()
    lax.fori_loop(0, n_outer, outer, ())

    plsc.subcore_barrier()

    def blk(b, _):
      rb0 = r0 + b * RB
      pltpu.sync_copy(out_sh.at[pl.ds(rb0, RB)], vbuf)
      pltpu.sync_copy(dinv_hbm.at[pl.ds(noff + rb0, RB)],
                      dv_v.at[pl.ds(0, RB)])

      def rowf(r, _):
        d = dv_v[pl.ds(r, L)][0]
        d2 = d * d
        for m in range(D // L):
          vbuf[r, pl.ds(m * L, L)] *= d2
        return ()
      lax.fori_loop(0, RB, rowf, ())
      pltpu.sync_copy(vbuf, gn_hbm.at[pl.ds(noff + rb0, RB)])
      return ()
    lax.fori_loop(0, ROWS // RB, blk, ())

  return layer_kernel


def _pq_body(x, g1, g2, g3, dinv, w1a, w1b, b1, p, q):
  hp = jax.lax.Precision.HIGHEST
  acc = x[:] + (g1[:] + g2[:] + g3[:]) / dinv[:][:, None]
  ne = acc * 0.25
  p[:] = jnp.dot(ne, w1a[:], precision=hp, preferred_element_type=f32) + b1[:]
  q[:] = jnp.dot(ne, w1b[:], precision=hp, preferred_element_type=f32)


def _pq_call(x, g1, g2, g3, dinv, w1a, w1b, b1):
  return pl.pallas_call(
      _pq_body,
      out_shape=(jax.ShapeDtypeStruct((2 * NP, D), f32),
                 jax.ShapeDtypeStruct((2 * NP, D), f32)),
  )(x, g1, g2, g3, dinv, w1a, w1b, b1)


def _make_logits_kernel(E):
  Et = E // NS
  n_chunks = Et // CHUNK
  n_outer = (n_chunks - 1) // 2

  @functools.partial(
      pl.kernel,
      out_type=jax.ShapeDtypeStruct((2 * E,), f32),
      mesh=_MESH,
      scratch_types=[
          pltpu.VMEM((n_chunks, CHUNK), i32),
          pltpu.VMEM((n_chunks, CHUNK), i32),
          pltpu.VMEM((2, CHUNK, D), f32),
          pltpu.VMEM((2, CHUNK, D), f32),
          pltpu.VMEM((D,), f32),
          pltpu.VMEM((L,), f32),
          pltpu.VMEM((Et,), f32),
          pltpu.SemaphoreType.DMA,
          pltpu.SemaphoreType.DMA,
          pltpu.SemaphoreType.DMA,
          pltpu.SemaphoreType.DMA,
      ],
  )
  def logits_kernel(p_hbm, q_hbm, src_hbm, dst_hbm, w2_hbm, b2_hbm, out_hbm,
                    idxa_v, idxb_v, bufp, bufq, w2_v, b2_v, out_v,
                    semp0, semq0, semp1, semq1):
    c = lax.axis_index("c")
    s = lax.axis_index("s")
    semp = (semp0, semp1)
    semq = (semq0, semq1)
    pltpu.sync_copy(src_hbm.at[c * NS + s], idxa_v)
    pltpu.sync_copy(dst_hbm.at[c * NS + s], idxb_v)
    pltpu.sync_copy(w2_hbm, w2_v)
    pltpu.sync_copy(b2_hbm, b2_v)
    w2vecs = [w2_v[pl.ds(m * L, L)] for m in range(D // L)]
    b2s = b2_v[...][0]
    lane = lax.iota(i32, L)

    def compute_chunk(ch, b):
      bp = bufp.at[b]
      bq = bufq.at[b]
      def group(jg, _):
        def edge(jj, ovec):
          j = jg * L + jj
          acc = jnp.zeros((L,), f32)
          for m in range(D // L):
            v = bp[j, pl.ds(m * L, L)] + bq[j, pl.ds(m * L, L)]
            v = jnp.maximum(v, 0.0)
            acc = acc + v * w2vecs[m]
          for sh in (8, 4, 2, 1):
            acc = acc + acc.at[lane ^ sh].get(mode="promise_in_bounds")
          return jnp.where(lane == jj, acc + b2s, ovec)
        ovec = lax.fori_loop(0, L, edge, jnp.zeros((L,), f32))
        out_v[pl.ds(ch * CHUNK + jg * L, L)] = ovec
        return ()
      lax.fori_loop(0, CHUNK // L, group, ())

    def issue(ch, b):
      pltpu.async_copy(p_hbm.at[idxa_v.at[ch]], bufp.at[b], semp[b])
      pltpu.async_copy(q_hbm.at[idxb_v.at[ch]], bufq.at[b], semq[b])

    def drain(b):
      pltpu.make_async_copy(p_hbm.at[pl.ds(0, CHUNK)], bufp.at[b],
                            semp[b]).wait()
      pltpu.make_async_copy(q_hbm.at[pl.ds(0, CHUNK)], bufq.at[b],
                            semq[b]).wait()

    issue(0, 0)
    issue(1, 1)

    def outer(g, _):
      for b in range(2):
        ch = g * 2 + b
        drain(b)
        compute_chunk(ch, b)
        nxt = ch + 2
        @pl.when(nxt < n_chunks)
        def _():
          issue(nxt, b)
      return ()
    lax.fori_loop(0, n_outer, outer, ())
    drain(0)
    compute_chunk(n_chunks - 1, 0)

    pltpu.sync_copy(out_v, out_hbm.at[pl.ds(c * E + s * Et, Et)])

  return logits_kernel


def kernel(user_emb, item_emb, W1, b1, W2, b2, trainUser, trainItem):
  E = trainUser.shape[0]
  n_users = user_emb.shape[0]
  n_items = item_emb.shape[0]
  assert E % (NS * CHUNK) == 0 and n_users <= NP and n_items <= NP

  tu = trainUser.astype(i32)
  ti = trainItem.astype(i32)
  all_dst = jnp.concatenate([ti, tu])
  n_ch = E // (NS * CHUNK)
  src_g2 = jnp.concatenate([tu, ti + NP]).reshape(NC * NS, n_ch, CHUNK)
  dst_l2 = all_dst.reshape(NC * NS, n_ch, CHUNK)
  xu = jnp.pad(user_emb, ((0, NP - n_users), (0, 0)))
  xi = jnp.pad(item_emb, ((0, NP - n_items), (0, 0)))
  x = jnp.concatenate([xu, xi], axis=0)

  deg = _make_deg_kernel(E)(dst_l2)
  dinv, g = _scale_call(deg, x)

  layer = _make_layer_kernel(E)
  g1 = layer(g, dinv, src_g2, dst_l2)
  g2 = layer(g1, dinv, src_g2, dst_l2)
  g3 = layer(g2, dinv, src_g2, dst_l2)

  p, q = _pq_call(x, g1, g2, g3, dinv, W1[:D], W1[D:], b1.reshape(1, D))

  w2 = W2[:, 0]
  b2p = jnp.pad(b2, (0, L - 1))
  dst_g2 = jnp.concatenate([ti + NP, tu]).reshape(NC * NS, n_ch, CHUNK)
  logits = _make_logits_kernel(E)(p, q, src_g2, dst_g2, w2, b2p)

  x0 = jnp.concatenate([user_emb, item_emb], axis=0)
  src = jnp.concatenate([tu, ti + n_users])
  dst = jnp.concatenate([ti + n_users, tu])
  edge_index = jnp.stack([src, dst])
  return (lax.stop_gradient(x0), edge_index,
          lax.stop_gradient(logits.reshape(2 * E, 1)))
